# baseline (speedup 1.0000x reference)
import functools

import jax
import jax.numpy as jnp
from jax import lax
from jax.experimental import pallas as pl
from jax.experimental.pallas import tpu as pltpu
from jax.experimental.pallas import tpu_sc as plsc

F = 128
B = 2048
S1 = 16
S2 = 16
NW = 32

R2 = (B * S1 * S2) // NW
R1 = (B * S1) // NW
R0 = B // NW
CHUNK = 256
GROUPS = CHUNK // 16


def _group_mean_rows(rows_v, acc_v, n_groups):
    nj = F // 16

    def body(r, carry):
        base = r * 16
        accs = [rows_v[base, pl.ds(j * 16, 16)] for j in range(nj)]
        for i in range(1, 16):
            vals = [rows_v[base + i, pl.ds(j * 16, 16)] for j in range(nj)]
            accs = [a + v for a, v in zip(accs, vals)]
        for j in range(nj):
            acc_v[r, pl.ds(j * 16, 16)] = accs[j] * (1.0 / 16.0)
        return carry

    lax.fori_loop(0, n_groups, body, 0)


NBUF = 3


def _sc_body(f0_hbm, f1_hbm, f2_hbm, feat_hbm,
             feat0_hbm, feat1_hbm, x1m_hbm, x2m_hbm,
             idx_v, rows0_v, rows1_v, rows2_v, out0_v, out1_v, out2_v,
             sg0, sg1, sg2, sw0, sw1, sw2, sem):
    wid = lax.axis_index("s") * 2 + lax.axis_index("c")
    rows = (rows0_v, rows1_v, rows2_v)
    outs = (out0_v, out1_v, out2_v)
    sgs = (sg0, sg1, sg2)
    sws = (sw0, sw1, sw2)

    def fire(chunk, b):
        start = chunk * CHUNK
        pltpu.async_copy(feat_hbm.at[idx_v.at[pl.ds(start, 128)]],
                         rows[b].at[pl.ds(0, 128)], sgs[b])
        pltpu.async_copy(feat_hbm.at[idx_v.at[pl.ds(start + 128, 128)]],
                         rows[b].at[pl.ds(128, 128)], sgs[b])

    def drain_gather(b):
        pltpu.make_async_copy(feat_hbm.at[pl.ds(0, CHUNK)], rows[b],
                              sgs[b]).wait()

    def drain_write(b, hbm_ref):
        pltpu.make_async_copy(outs[b], hbm_ref.at[pl.ds(0, GROUPS)],
                              sws[b]).wait()

    pltpu.sync_copy(f2_hbm.at[pl.ds(wid * R2, R2)], idx_v)
    NCH2 = R2 // CHUNK
    for b in range(NBUF):
        fire(b, b)

    def step(chunk, b):
        drain_gather(b)

        @pl.when(chunk >= NBUF)
        def _():
            drain_write(b, x2m_hbm)

        _group_mean_rows(rows[b], outs[b], GROUPS)

        @pl.when(chunk + NBUF < NCH2)
        def _():
            fire(chunk + NBUF, b)

        pltpu.async_copy(
            outs[b],
            x2m_hbm.at[pl.ds(wid * (R2 // 16) + chunk * GROUPS, GROUPS)],
            sws[b])

    def ring(g, carry):
        for b in range(NBUF):
            step(NBUF * g + b, b)
        return carry

    n_full = NCH2 // NBUF
    lax.fori_loop(0, n_full, ring, 0)
    for c in range(n_full * NBUF, NCH2):
        step(jnp.int32(c), c % NBUF)
    for b in range(NBUF):
        drain_write(b, x2m_hbm)

    pltpu.sync_copy(f1_hbm.at[pl.ds(wid * R1, R1)], idx_v.at[pl.ds(0, R1)])

    pltpu.sync_copy(f1_hbm.at[pl.ds(wid * R1, R1)], idx_v.at[pl.ds(0, R1)])
    NCH1 = R1 // CHUNK
    for b in range(NBUF):
        fire(b, b)

    def drain_pass():
        pltpu.make_async_copy(rows[0], feat1_hbm.at[pl.ds(0, CHUNK)],
                              sem).wait()

    for c in range(NCH1):
        b = c % NBUF
        drain_gather(b)
        if c >= NBUF:
            drain_write(b, x1m_hbm)
        _group_mean_rows(rows[b], outs[b], GROUPS)
        pltpu.async_copy(
            outs[b], x1m_hbm.at[pl.ds(wid * (R1 // 16) + c * GROUPS, GROUPS)],
            sws[b])
        pltpu.async_copy(
            rows[b], feat1_hbm.at[pl.ds(wid * R1 + c * CHUNK, CHUNK)], sem)
        if c + NBUF < NCH1:
            drain_pass()
            fire(c + NBUF, b)

    for _ in range(min(NBUF, NCH1)):
        drain_pass()
    for b in range(min(NBUF, NCH1)):
        drain_write(b, x1m_hbm)

    pltpu.sync_copy(f0_hbm.at[pl.ds(wid * R0, R0)], idx_v.at[pl.ds(0, R0)])
    pltpu.async_copy(
        feat_hbm.at[idx_v.at[pl.ds(0, R0)]],
        rows0_v.at[pl.ds(0, R0)], sem).wait()
    pltpu.sync_copy(rows0_v.at[pl.ds(0, R0)],
                    feat0_hbm.at[pl.ds(wid * R0, R0)])


def _sc_gather(f0, f1, f2, feature_matrix):
    mesh = plsc.VectorSubcoreMesh(core_axis_name="c", subcore_axis_name="s", num_cores=2, num_subcores=16)
    fn = pl.kernel(
        _sc_body,
        mesh=mesh,
        out_type=[
            jax.ShapeDtypeStruct((B, F), jnp.float32),
            jax.ShapeDtypeStruct((B * S1, F), jnp.float32),
            jax.ShapeDtypeStruct((B, F), jnp.float32),
            jax.ShapeDtypeStruct((B * S1, F), jnp.float32),
        ],
        scratch_types=[
            pltpu.VMEM((R2,), jnp.int32),
            pltpu.VMEM((CHUNK, F), jnp.float32),
            pltpu.VMEM((CHUNK, F), jnp.float32),
            pltpu.VMEM((CHUNK, F), jnp.float32),
            pltpu.VMEM((GROUPS, F), jnp.float32),
            pltpu.VMEM((GROUPS, F), jnp.float32),
            pltpu.VMEM((GROUPS, F), jnp.float32),
            pltpu.SemaphoreType.DMA,
            pltpu.SemaphoreType.DMA,
            pltpu.SemaphoreType.DMA,
            pltpu.SemaphoreType.DMA,
            pltpu.SemaphoreType.DMA,
            pltpu.SemaphoreType.DMA,
            pltpu.SemaphoreType.DMA,
        ],
    )
    return fn(f0, f1, f2, feature_matrix)


def _tc_body(feat0_ref, x1m_ref, feat1_ref, x2m_ref, w1_ref, w2_ref, out_ref):
    w1a = w1_ref[:F, :]
    w1b = w1_ref[F:, :]
    h1 = jnp.maximum(
        jnp.dot(feat1_ref[...], w1a, preferred_element_type=jnp.float32)
        + jnp.dot(x2m_ref[...], w1b, preferred_element_type=jnp.float32), 0.0)
    bm = out_ref.shape[0]
    h1m = h1.reshape(bm, S1, h1.shape[1]).mean(axis=1)
    h0 = jnp.maximum(
        jnp.dot(feat0_ref[...], w1a, preferred_element_type=jnp.float32)
        + jnp.dot(x1m_ref[...], w1b, preferred_element_type=jnp.float32), 0.0)
    out_ref[...] = jnp.maximum(
        jnp.dot(h0, w2_ref[:h0.shape[1], :], preferred_element_type=jnp.float32)
        + jnp.dot(h1m, w2_ref[h0.shape[1]:, :],
                  preferred_element_type=jnp.float32), 0.0)


def _tc_dense(feat0, x1m, feat1, x2m, W1, W2):
    BM = 256
    grid = B // BM
    return pl.pallas_call(
        _tc_body,
        grid=(grid,),
        in_specs=[
            pl.BlockSpec((BM, F), lambda i: (i, 0)),
            pl.BlockSpec((BM, F), lambda i: (i, 0)),
            pl.BlockSpec((BM * S1, F), lambda i: (i, 0)),
            pl.BlockSpec((BM * S1, F), lambda i: (i, 0)),
            pl.BlockSpec((2 * F, W1.shape[1]), lambda i: (0, 0)),
            pl.BlockSpec((2 * W1.shape[1], W2.shape[1]), lambda i: (0, 0)),
        ],
        out_specs=pl.BlockSpec((BM, W2.shape[1]), lambda i: (i, 0)),
        out_shape=jax.ShapeDtypeStruct((B, W2.shape[1]), jnp.float32),
    )(feat0, x1m, feat1, x2m, W1, W2)


def kernel(forest_0, forest_1, forest_2, feature_matrix, W1, W2):
    f0 = forest_0.astype(jnp.int32)
    f1 = forest_1.reshape(-1).astype(jnp.int32)
    f2 = forest_2.reshape(-1).astype(jnp.int32)
    feat0, feat1, x1m, x2m = _sc_gather(f0, f1, f2, feature_matrix)
    return _tc_dense(feat0, x1m, feat1, x2m, W1, W2)


# speedup vs baseline: 1.0013x; 1.0013x over previous
import functools

import jax
import jax.numpy as jnp
from jax import lax
from jax.experimental import pallas as pl
from jax.experimental.pallas import tpu as pltpu
from jax.experimental.pallas import tpu_sc as plsc

F = 128
B = 2048
S1 = 16
S2 = 16
NW = 32

R2 = (B * S1 * S2) // NW
R1 = (B * S1) // NW
R0 = B // NW
CHUNK = 128
GROUPS = CHUNK // 16


def _group_mean_rows(rows_v, acc_v, n_groups):
    nj = F // 16

    def body(r, carry):
        base = r * 16
        accs = [rows_v[base, pl.ds(j * 16, 16)] for j in range(nj)]
        for i in range(1, 16):
            vals = [rows_v[base + i, pl.ds(j * 16, 16)] for j in range(nj)]
            accs = [a + v for a, v in zip(accs, vals)]
        for j in range(nj):
            acc_v[r, pl.ds(j * 16, 16)] = accs[j] * (1.0 / 16.0)
        return carry

    lax.fori_loop(0, n_groups, body, 0)


NBUF = 6


def _sc_body(f0_hbm, f1_hbm, f2_hbm, feat_hbm,
             feat0_hbm, feat1_hbm, x1m_hbm, x2m_hbm,
             idx_v, *bufs):
    wid = lax.axis_index("s") * 2 + lax.axis_index("c")
    rows = bufs[0:NBUF]
    outs = bufs[NBUF:2 * NBUF]
    sgs = bufs[2 * NBUF:3 * NBUF]
    sws = bufs[3 * NBUF:4 * NBUF]
    sem = bufs[4 * NBUF]

    def fire(chunk, b):
        start = chunk * CHUNK
        for h in range(CHUNK // 128):
            pltpu.async_copy(
                feat_hbm.at[idx_v.at[pl.ds(start + h * 128, 128)]],
                rows[b].at[pl.ds(h * 128, 128)], sgs[b])

    def drain_gather(b):
        pltpu.make_async_copy(feat_hbm.at[pl.ds(0, CHUNK)], rows[b],
                              sgs[b]).wait()

    def drain_write(b, hbm_ref):
        pltpu.make_async_copy(outs[b], hbm_ref.at[pl.ds(0, GROUPS)],
                              sws[b]).wait()

    pltpu.sync_copy(f2_hbm.at[pl.ds(wid * R2, R2)], idx_v)
    NCH2 = R2 // CHUNK
    for b in range(NBUF):
        fire(b, b)

    def step(chunk, b):
        drain_gather(b)

        @pl.when(chunk >= NBUF)
        def _():
            drain_write(b, x2m_hbm)

        _group_mean_rows(rows[b], outs[b], GROUPS)

        @pl.when(chunk + NBUF < NCH2)
        def _():
            fire(chunk + NBUF, b)

        pltpu.async_copy(
            outs[b],
            x2m_hbm.at[pl.ds(wid * (R2 // 16) + chunk * GROUPS, GROUPS)],
            sws[b])

    def ring(g, carry):
        for b in range(NBUF):
            step(NBUF * g + b, b)
        return carry

    n_full = NCH2 // NBUF
    lax.fori_loop(0, n_full, ring, 0)
    for c in range(n_full * NBUF, NCH2):
        step(jnp.int32(c), c % NBUF)
    for b in range(NBUF):
        drain_write(b, x2m_hbm)

    pltpu.sync_copy(f1_hbm.at[pl.ds(wid * R1, R1)], idx_v.at[pl.ds(0, R1)])
    NCH1 = R1 // CHUNK
    for b in range(NBUF):
        fire(b, b)

    def drain_pass():
        pltpu.make_async_copy(rows[0], feat1_hbm.at[pl.ds(0, CHUNK)],
                              sem).wait()

    for c in range(NCH1):
        b = c % NBUF
        drain_gather(b)
        if c >= NBUF:
            drain_write(b, x1m_hbm)
        _group_mean_rows(rows[b], outs[b], GROUPS)
        pltpu.async_copy(
            outs[b], x1m_hbm.at[pl.ds(wid * (R1 // 16) + c * GROUPS, GROUPS)],
            sws[b])
        pltpu.async_copy(
            rows[b], feat1_hbm.at[pl.ds(wid * R1 + c * CHUNK, CHUNK)], sem)
        if c + NBUF < NCH1:
            drain_pass()
            fire(c + NBUF, b)

    for _ in range(min(NBUF, NCH1)):
        drain_pass()
    for b in range(min(NBUF, NCH1)):
        drain_write(b, x1m_hbm)

    pltpu.sync_copy(f0_hbm.at[pl.ds(wid * R0, R0)], idx_v.at[pl.ds(0, R0)])
    pltpu.async_copy(
        feat_hbm.at[idx_v.at[pl.ds(0, R0)]],
        rows[0].at[pl.ds(0, R0)], sem).wait()
    pltpu.sync_copy(rows[0].at[pl.ds(0, R0)],
                    feat0_hbm.at[pl.ds(wid * R0, R0)])


def _sc_gather(f0, f1, f2, feature_matrix):
    mesh = plsc.VectorSubcoreMesh(core_axis_name="c", subcore_axis_name="s", num_cores=2, num_subcores=16)
    fn = pl.kernel(
        _sc_body,
        mesh=mesh,
        out_type=[
            jax.ShapeDtypeStruct((B, F), jnp.float32),
            jax.ShapeDtypeStruct((B * S1, F), jnp.float32),
            jax.ShapeDtypeStruct((B, F), jnp.float32),
            jax.ShapeDtypeStruct((B * S1, F), jnp.float32),
        ],
        scratch_types=(
            [pltpu.VMEM((R2,), jnp.int32)]
            + [pltpu.VMEM((CHUNK, F), jnp.float32)] * NBUF
            + [pltpu.VMEM((GROUPS, F), jnp.float32)] * NBUF
            + [pltpu.SemaphoreType.DMA] * (2 * NBUF + 1)
        ),
    )
    return fn(f0, f1, f2, feature_matrix)


def _tc_body(feat0_ref, x1m_ref, feat1_ref, x2m_ref, w1_ref, w2_ref, out_ref):
    w1a = w1_ref[:F, :]
    w1b = w1_ref[F:, :]
    h1 = jnp.maximum(
        jnp.dot(feat1_ref[...], w1a, preferred_element_type=jnp.float32)
        + jnp.dot(x2m_ref[...], w1b, preferred_element_type=jnp.float32), 0.0)
    bm = out_ref.shape[0]
    h1m = h1.reshape(bm, S1, h1.shape[1]).mean(axis=1)
    h0 = jnp.maximum(
        jnp.dot(feat0_ref[...], w1a, preferred_element_type=jnp.float32)
        + jnp.dot(x1m_ref[...], w1b, preferred_element_type=jnp.float32), 0.0)
    out_ref[...] = jnp.maximum(
        jnp.dot(h0, w2_ref[:h0.shape[1], :], preferred_element_type=jnp.float32)
        + jnp.dot(h1m, w2_ref[h0.shape[1]:, :],
                  preferred_element_type=jnp.float32), 0.0)


def _tc_dense(feat0, x1m, feat1, x2m, W1, W2):
    BM = 256
    grid = B // BM
    return pl.pallas_call(
        _tc_body,
        grid=(grid,),
        in_specs=[
            pl.BlockSpec((BM, F), lambda i: (i, 0)),
            pl.BlockSpec((BM, F), lambda i: (i, 0)),
            pl.BlockSpec((BM * S1, F), lambda i: (i, 0)),
            pl.BlockSpec((BM * S1, F), lambda i: (i, 0)),
            pl.BlockSpec((2 * F, W1.shape[1]), lambda i: (0, 0)),
            pl.BlockSpec((2 * W1.shape[1], W2.shape[1]), lambda i: (0, 0)),
        ],
        out_specs=pl.BlockSpec((BM, W2.shape[1]), lambda i: (i, 0)),
        out_shape=jax.ShapeDtypeStruct((B, W2.shape[1]), jnp.float32),
    )(feat0, x1m, feat1, x2m, W1, W2)


def kernel(forest_0, forest_1, forest_2, feature_matrix, W1, W2):
    f0 = forest_0.astype(jnp.int32)
    f1 = forest_1.reshape(-1).astype(jnp.int32)
    f2 = forest_2.reshape(-1).astype(jnp.int32)
    feat0, feat1, x1m, x2m = _sc_gather(f0, f1, f2, feature_matrix)
    return _tc_dense(feat0, x1m, feat1, x2m, W1, W2)
